# Initial kernel scaffold; baseline (speedup 1.0000x reference)
import jax
import jax.numpy as jnp
from jax.experimental import pallas as pl


def kernel(x, table):
    raise NotImplementedError("write your pallas kernel here")


# speedup vs baseline: 1.4818x; 1.4818x over previous
import functools

import jax
import jax.numpy as jnp
from jax import lax
from jax.experimental import pallas as pl
from jax.experimental.pallas import tpu as pltpu
from jax.experimental.pallas import tpu_sc as plsc

B = 16384
F = 26
VOCAB = 1000000

NC = 2
NS = 16
NW = NC * NS
CHUNK = B // NW
NIDX = CHUNK * F
L = 16


def _body(tf_hbm, xr_hbm, out_hbm, idx_v, vals_v, out_v, sem):
    wid = lax.axis_index("s") * NC + lax.axis_index("c")

    pltpu.sync_copy(xr_hbm.at[wid], idx_v)

    pltpu.async_copy(tf_hbm.at[idx_v], vals_v, sem).wait()

    def g_body(g, _):
        base = g * L
        acc = jnp.zeros((L,), jnp.float32)
        for f in range(F):
            acc = acc + vals_v[pl.ds(f * CHUNK + base, L)]
        out_v[pl.ds(base, L)] = 1.0 / (1.0 + jnp.exp(-acc))
        return _

    lax.fori_loop(0, CHUNK // L, g_body, None)

    pltpu.sync_copy(out_v, out_hbm.at[pl.ds(wid * CHUNK, CHUNK)])


_sc_call = functools.partial(
    pl.kernel,
    out_type=jax.ShapeDtypeStruct((B,), jnp.float32),
    mesh=plsc.VectorSubcoreMesh(
        core_axis_name="c", subcore_axis_name="s",
        num_cores=NC, num_subcores=NS,
    ),
    scratch_types=[
        pltpu.VMEM((NIDX,), jnp.int32),
        pltpu.VMEM((NIDX,), jnp.float32),
        pltpu.VMEM((CHUNK,), jnp.float32),
        pltpu.SemaphoreType.DMA,
    ],
)(_body)


@jax.jit
def kernel(x, table):
    xr = x.reshape(NW, CHUNK, F).transpose(0, 2, 1).reshape(NW, NIDX)
    tf = table.reshape(VOCAB)
    return _sc_call(tf, xr)
